# baseline (speedup 1.0000x reference)
import functools

import jax
import jax.numpy as jnp
from jax import lax
from jax.experimental import pallas as pl
from jax.experimental.pallas import tpu as pltpu
from jax.experimental.pallas import tpu_sc as plsc

_B = 16384
_F = 2048
_L = 16
_NC, _NS = 2, 16
_NW = _NC * _NS
_RPW = _B // _NW
_R = 8
_NBUF = 2
_CH = _RPW // _R
_G = _CH // _NBUF


def _sc_permute_body(x_hbm, perm_hbm, out_hbm, perm_v, in_bufs, out_bufs,
                     in_sems, out_sems):
    wid = lax.axis_index("s") * _NC + lax.axis_index("c")
    row0 = wid * _RPW
    pltpu.sync_copy(perm_hbm, perm_v)

    def in_desc(c, b):
        return pltpu.make_async_copy(
            x_hbm.at[pl.ds((row0 + c * _R) * _F, _R * _F)],
            in_bufs[b], in_sems[b])

    def out_desc(c, b):
        return pltpu.make_async_copy(
            out_bufs[b],
            out_hbm.at[pl.ds((row0 + c * _R) * _F, _R * _F)],
            out_sems[b])

    for b in range(_NBUF):
        in_desc(b, b).start()

    def group(g, _):
        for b in range(_NBUF):
            c = g * _NBUF + b
            in_desc(c, b).wait()

            @pl.when(g > 0)
            def _wait_prev_out():
                out_desc(c - _NBUF, b).wait()

            def col(j, carry):
                idx = perm_v[pl.ds(j * _L, _L)]
                for r in range(_R):
                    vals = plsc.load_gather(in_bufs[b], [idx + r * _F])
                    out_bufs[b][pl.ds(r * _F + j * _L, _L)] = vals
                return carry

            lax.fori_loop(0, _F // _L, col, 0, unroll=2)
            out_desc(c, b).start()

            @pl.when(c + _NBUF < _CH)
            def _start_next_in():
                in_desc(c + _NBUF, b).start()
        return 0

    lax.fori_loop(0, _G, group, 0)
    for b in range(_NBUF):
        out_desc(_CH - _NBUF + b, b).wait()


_sc_permute = functools.partial(
    pl.kernel,
    out_type=jax.ShapeDtypeStruct((_B * _F,), jnp.float32),
    mesh=plsc.VectorSubcoreMesh(core_axis_name="c", subcore_axis_name="s",
                                num_cores=_NC, num_subcores=_NS),
    scratch_types=[
        pltpu.VMEM((_F,), jnp.int32),
        [pltpu.VMEM((_R * _F,), jnp.float32) for _ in range(_NBUF)],
        [pltpu.VMEM((_R * _F,), jnp.float32) for _ in range(_NBUF)],
        [pltpu.SemaphoreType.DMA for _ in range(_NBUF)],
        [pltpu.SemaphoreType.DMA for _ in range(_NBUF)],
    ],
    compiler_params=pltpu.CompilerParams(needs_layout_passes=False),
)(_sc_permute_body)


def kernel(x, perm):
    out_flat = _sc_permute(x.reshape(-1), perm)
    return out_flat.reshape(_B, _F)


# speedup vs baseline: 1.5751x; 1.5751x over previous
import functools

import jax
import jax.numpy as jnp
from jax import lax
from jax.experimental import pallas as pl
from jax.experimental.pallas import tpu as pltpu
from jax.experimental.pallas import tpu_sc as plsc

_B = 16384
_F = 2048
_L = 16
_NC, _NS = 2, 16
_NW = _NC * _NS
_RPW = _B // _NW
_R = 8
_NBUF = 2
_CH = _RPW // _R
_G = _CH // _NBUF


def _sc_permute_body(x_hbm, perm_hbm, out_hbm, perm_v, in_bufs, out_bufs,
                     in_sems, out_sems):
    wid = lax.axis_index("s") * _NC + lax.axis_index("c")
    row0 = wid * _RPW
    pltpu.sync_copy(perm_hbm, perm_v)

    def in_desc(c, b):
        return pltpu.make_async_copy(
            x_hbm.at[pl.ds(row0 + c * _R, _R), :], in_bufs[b], in_sems[b])

    def out_desc(c, b):
        return pltpu.make_async_copy(
            out_bufs[b], out_hbm.at[pl.ds(row0 + c * _R, _R), :], out_sems[b])

    for b in range(_NBUF):
        in_desc(b, b).start()

    def group(g, _):
        for b in range(_NBUF):
            c = g * _NBUF + b
            in_desc(c, b).wait()

            @pl.when(g > 0)
            def _wait_prev_out():
                out_desc(c - _NBUF, b).wait()

            def col(j, carry):
                idx = perm_v[pl.ds(j * _L, _L)]
                for r in range(_R):
                    vals = plsc.load_gather(
                        in_bufs[b], [jnp.full((_L,), r, jnp.int32), idx])
                    out_bufs[b][r, pl.ds(j * _L, _L)] = vals
                return carry

            lax.fori_loop(0, _F // _L, col, 0, unroll=2)
            out_desc(c, b).start()

            @pl.when(c + _NBUF < _CH)
            def _start_next_in():
                in_desc(c + _NBUF, b).start()
        return 0

    lax.fori_loop(0, _G, group, 0)
    for b in range(_NBUF):
        out_desc(_CH - _NBUF + b, b).wait()


_sc_permute = functools.partial(
    pl.kernel,
    out_type=jax.ShapeDtypeStruct((_B, _F), jnp.float32),
    mesh=plsc.VectorSubcoreMesh(core_axis_name="c", subcore_axis_name="s",
                                num_cores=_NC, num_subcores=_NS),
    scratch_types=[
        pltpu.VMEM((_F,), jnp.int32),
        [pltpu.VMEM((_R, _F), jnp.float32) for _ in range(_NBUF)],
        [pltpu.VMEM((_R, _F), jnp.float32) for _ in range(_NBUF)],
        [pltpu.SemaphoreType.DMA for _ in range(_NBUF)],
        [pltpu.SemaphoreType.DMA for _ in range(_NBUF)],
    ],
    compiler_params=pltpu.CompilerParams(needs_layout_passes=False),
)(_sc_permute_body)


def kernel(x, perm):
    return _sc_permute(x, perm)


# speedup vs baseline: 4.5986x; 2.9196x over previous
import functools

import jax
import jax.numpy as jnp
from jax import lax
from jax.experimental import pallas as pl
from jax.experimental.pallas import tpu as pltpu
from jax.experimental.pallas import tpu_sc as plsc

_B = 16384
_F = 2048
_L = 16
_NC, _NS = 2, 16
_NW = _NC * _NS
_RPW = _B // _NW
_R = 8
_NBUF = 2
_CH = _RPW // _R
_G = _CH // _NBUF


def _sc_permute_body(x_hbm, perm_hbm, out_hbm, perm_v, in_bufs, out_bufs,
                     in_sems, out_sems):
    wid = lax.axis_index("s") * _NC + lax.axis_index("c")
    row0 = wid * _RPW
    pltpu.sync_copy(perm_hbm, perm_v)

    def in_desc(c, b):
        return pltpu.make_async_copy(
            x_hbm.at[pl.ds(row0 + c * _R, _R), :], in_bufs[b], in_sems[b])

    def out_desc(c, b):
        return pltpu.make_async_copy(
            out_bufs[b], out_hbm.at[pl.ds(row0 + c * _R, _R), :], out_sems[b])

    for b in range(_NBUF):
        in_desc(b, b).start()

    def group(g, _):
        for b in range(_NBUF):
            c = g * _NBUF + b
            in_desc(c, b).wait()

            @pl.when(g > 0)
            def _wait_prev_out():
                out_desc(c - _NBUF, b).wait()

            @plsc.parallel_loop(0, _F // _L, unroll=2)
            def _col(j):
                idx = perm_v[pl.ds(j * _L, _L)]
                vals = [
                    plsc.load_gather(
                        in_bufs[b], [jnp.full((_L,), r, jnp.int32), idx])
                    for r in range(_R)
                ]
                for r in range(_R):
                    out_bufs[b][r, pl.ds(j * _L, _L)] = vals[r]
            out_desc(c, b).start()

            @pl.when(c + _NBUF < _CH)
            def _start_next_in():
                in_desc(c + _NBUF, b).start()
        return 0

    lax.fori_loop(0, _G, group, 0)
    for b in range(_NBUF):
        out_desc(_CH - _NBUF + b, b).wait()


_sc_permute = functools.partial(
    pl.kernel,
    out_type=jax.ShapeDtypeStruct((_B, _F), jnp.float32),
    mesh=plsc.VectorSubcoreMesh(core_axis_name="c", subcore_axis_name="s",
                                num_cores=_NC, num_subcores=_NS),
    scratch_types=[
        pltpu.VMEM((_F,), jnp.int32),
        [pltpu.VMEM((_R, _F), jnp.float32) for _ in range(_NBUF)],
        [pltpu.VMEM((_R, _F), jnp.float32) for _ in range(_NBUF)],
        [pltpu.SemaphoreType.DMA for _ in range(_NBUF)],
        [pltpu.SemaphoreType.DMA for _ in range(_NBUF)],
    ],
    compiler_params=pltpu.CompilerParams(needs_layout_passes=False),
)(_sc_permute_body)


def kernel(x, perm):
    return _sc_permute(x, perm)


# speedup vs baseline: 4.8953x; 1.0645x over previous
import functools

import jax
import jax.numpy as jnp
from jax import lax
from jax.experimental import pallas as pl
from jax.experimental.pallas import tpu as pltpu
from jax.experimental.pallas import tpu_sc as plsc

_B = 16384
_F = 2048
_L = 16
_NC, _NS = 2, 16
_NW = _NC * _NS
_RPW = _B // _NW
_R = 8
_NBUF = 3
_CH = _RPW // _R
_G = (_CH - 1) // _NBUF


def _sc_permute_body(x_hbm, perm_hbm, out_hbm, perm_v, in_bufs, out_bufs,
                     in_sems, out_sems):
    wid = lax.axis_index("s") * _NC + lax.axis_index("c")
    row0 = wid * _RPW
    pltpu.sync_copy(perm_hbm, perm_v)

    def in_desc(c, b):
        return pltpu.make_async_copy(
            x_hbm.at[pl.ds(row0 + c * _R, _R), :], in_bufs[b], in_sems[b])

    def out_desc(c, b):
        return pltpu.make_async_copy(
            out_bufs[b], out_hbm.at[pl.ds(row0 + c * _R, _R), :], out_sems[b])

    def permute_chunk(b):
        @plsc.parallel_loop(0, _F // _L, unroll=2)
        def _col(j):
            idx = perm_v[pl.ds(j * _L, _L)]
            vals = [
                plsc.load_gather(
                    in_bufs[b], [jnp.full((_L,), r, jnp.int32), idx])
                for r in range(_R)
            ]
            for r in range(_R):
                out_bufs[b][r, pl.ds(j * _L, _L)] = vals[r]

    for b in range(_NBUF):
        in_desc(b, b).start()

    def group(g, _):
        for b in range(_NBUF):
            c = g * _NBUF + b
            in_desc(c, b).wait()

            @pl.when(g > 0)
            def _wait_prev_out():
                out_desc(c - _NBUF, b).wait()

            permute_chunk(b)
            out_desc(c, b).start()

            @pl.when(c + _NBUF < _CH)
            def _start_next_in():
                in_desc(c + _NBUF, b).start()
        return 0

    lax.fori_loop(0, _G, group, 0)

    last = _CH - 1
    in_desc(last, 0).wait()
    out_desc(last - _NBUF, 0).wait()
    permute_chunk(0)
    out_desc(last, 0).start()
    out_desc(last - 2, 1).wait()
    out_desc(last - 1, 2).wait()
    out_desc(last, 0).wait()


_sc_permute = functools.partial(
    pl.kernel,
    out_type=jax.ShapeDtypeStruct((_B, _F), jnp.float32),
    mesh=plsc.VectorSubcoreMesh(core_axis_name="c", subcore_axis_name="s",
                                num_cores=_NC, num_subcores=_NS),
    scratch_types=[
        pltpu.VMEM((_F,), jnp.int32),
        [pltpu.VMEM((_R, _F), jnp.float32) for _ in range(_NBUF)],
        [pltpu.VMEM((_R, _F), jnp.float32) for _ in range(_NBUF)],
        [pltpu.SemaphoreType.DMA for _ in range(_NBUF)],
        [pltpu.SemaphoreType.DMA for _ in range(_NBUF)],
    ],
    compiler_params=pltpu.CompilerParams(needs_layout_passes=False),
)(_sc_permute_body)


def kernel(x, perm):
    return _sc_permute(x, perm)


# speedup vs baseline: 4.8989x; 1.0007x over previous
import functools

import jax
import jax.numpy as jnp
from jax import lax
from jax.experimental import pallas as pl
from jax.experimental.pallas import tpu as pltpu
from jax.experimental.pallas import tpu_sc as plsc

_B = 16384
_F = 2048
_L = 16
_NC, _NS = 2, 16
_NW = _NC * _NS
_RPW = _B // _NW
_R = 8
_NBUF = 3
_CH = _RPW // _R
_G = (_CH - 1) // _NBUF


def _sc_permute_body(x_hbm, perm_hbm, out_hbm, perm_v, in_bufs, out_bufs,
                     in_sems, out_sems):
    wid = lax.axis_index("s") * _NC + lax.axis_index("c")
    row0 = wid * _RPW
    pltpu.sync_copy(perm_hbm, perm_v)

    def in_desc(c, b):
        return pltpu.make_async_copy(
            x_hbm.at[pl.ds(row0 + c * _R, _R), :], in_bufs[b], in_sems[b])

    def out_desc(c, b):
        return pltpu.make_async_copy(
            out_bufs[b], out_hbm.at[pl.ds(row0 + c * _R, _R), :], out_sems[b])

    def permute_chunk(b):
        @plsc.parallel_loop(0, _F // _L, unroll=4)
        def _col(j):
            idx = perm_v[pl.ds(j * _L, _L)]
            vals = [
                plsc.load_gather(
                    in_bufs[b], [jnp.full((_L,), r, jnp.int32), idx])
                for r in range(_R)
            ]
            for r in range(_R):
                out_bufs[b][r, pl.ds(j * _L, _L)] = vals[r]

    for b in range(_NBUF):
        in_desc(b, b).start()

    def group(g, _):
        for b in range(_NBUF):
            c = g * _NBUF + b
            in_desc(c, b).wait()

            @pl.when(g > 0)
            def _wait_prev_out():
                out_desc(c - _NBUF, b).wait()

            permute_chunk(b)
            out_desc(c, b).start()

            @pl.when(c + _NBUF < _CH)
            def _start_next_in():
                in_desc(c + _NBUF, b).start()
        return 0

    lax.fori_loop(0, _G, group, 0)

    last = _CH - 1
    in_desc(last, 0).wait()
    out_desc(last - _NBUF, 0).wait()
    permute_chunk(0)
    out_desc(last, 0).start()
    out_desc(last - 2, 1).wait()
    out_desc(last - 1, 2).wait()
    out_desc(last, 0).wait()


_sc_permute = functools.partial(
    pl.kernel,
    out_type=jax.ShapeDtypeStruct((_B, _F), jnp.float32),
    mesh=plsc.VectorSubcoreMesh(core_axis_name="c", subcore_axis_name="s",
                                num_cores=_NC, num_subcores=_NS),
    scratch_types=[
        pltpu.VMEM((_F,), jnp.int32),
        [pltpu.VMEM((_R, _F), jnp.float32) for _ in range(_NBUF)],
        [pltpu.VMEM((_R, _F), jnp.float32) for _ in range(_NBUF)],
        [pltpu.SemaphoreType.DMA for _ in range(_NBUF)],
        [pltpu.SemaphoreType.DMA for _ in range(_NBUF)],
    ],
    compiler_params=pltpu.CompilerParams(needs_layout_passes=False),
)(_sc_permute_body)


def kernel(x, perm):
    return _sc_permute(x, perm)


# speedup vs baseline: 4.9395x; 1.0083x over previous
import functools

import jax
import jax.numpy as jnp
from jax import lax
from jax.experimental import pallas as pl
from jax.experimental.pallas import tpu as pltpu
from jax.experimental.pallas import tpu_sc as plsc

_B = 16384
_F = 2048
_L = 16
_NC, _NS = 2, 16
_NW = _NC * _NS
_RPW = _B // _NW
_R = 8
_NBUF = 3
_CH = _RPW // _R
_G = (_CH - 1) // _NBUF


def _sc_permute_body(x_hbm, perm_hbm, out_hbm, perm_v, in_bufs, out_bufs,
                     in_sems, out_sems):
    wid = lax.axis_index("s") * _NC + lax.axis_index("c")
    row0 = wid * _RPW

    def in_desc(c, b):
        return pltpu.make_async_copy(
            x_hbm.at[pl.ds(row0 + c * _R, _R), :], in_bufs[b], in_sems[b])

    def out_desc(c, b):
        return pltpu.make_async_copy(
            out_bufs[b], out_hbm.at[pl.ds(row0 + c * _R, _R), :], out_sems[b])

    def permute_chunk(b):
        @plsc.parallel_loop(0, _F // _L, unroll=4)
        def _col(j):
            idx = perm_v[pl.ds(j * _L, _L)]
            vals = [
                plsc.load_gather(
                    in_bufs[b], [jnp.full((_L,), r, jnp.int32), idx])
                for r in range(_R)
            ]
            for r in range(_R):
                out_bufs[b][r, pl.ds(j * _L, _L)] = vals[r]

    for b in range(_NBUF):
        in_desc(b, b).start()
    pltpu.sync_copy(perm_hbm, perm_v)

    def group(g, _):
        for b in range(_NBUF):
            c = g * _NBUF + b
            in_desc(c, b).wait()

            @pl.when(g > 0)
            def _wait_prev_out():
                out_desc(c - _NBUF, b).wait()

            permute_chunk(b)
            out_desc(c, b).start()

            @pl.when(c + _NBUF < _CH)
            def _start_next_in():
                in_desc(c + _NBUF, b).start()
        return 0

    lax.fori_loop(0, _G, group, 0)

    last = _CH - 1
    in_desc(last, 0).wait()
    out_desc(last - _NBUF, 0).wait()
    permute_chunk(0)
    out_desc(last, 0).start()
    out_desc(last - 2, 1).wait()
    out_desc(last - 1, 2).wait()
    out_desc(last, 0).wait()


_sc_permute = functools.partial(
    pl.kernel,
    out_type=jax.ShapeDtypeStruct((_B, _F), jnp.float32),
    mesh=plsc.VectorSubcoreMesh(core_axis_name="c", subcore_axis_name="s",
                                num_cores=_NC, num_subcores=_NS),
    scratch_types=[
        pltpu.VMEM((_F,), jnp.int32),
        [pltpu.VMEM((_R, _F), jnp.float32) for _ in range(_NBUF)],
        [pltpu.VMEM((_R, _F), jnp.float32) for _ in range(_NBUF)],
        [pltpu.SemaphoreType.DMA for _ in range(_NBUF)],
        [pltpu.SemaphoreType.DMA for _ in range(_NBUF)],
    ],
    compiler_params=pltpu.CompilerParams(needs_layout_passes=False),
)(_sc_permute_body)


def kernel(x, perm):
    return _sc_permute(x, perm)
